# Initial kernel scaffold; baseline (speedup 1.0000x reference)
import jax
import jax.numpy as jnp
from jax.experimental import pallas as pl


def kernel(xyz, xyz_batch_cnt, pt_feature, bxyz, W1, b1):
    raise NotImplementedError("write your pallas kernel here")


# speedup vs baseline: 1.1078x; 1.1078x over previous
import functools

import jax
import jax.numpy as jnp
from jax.experimental import pallas as pl

_PILLAR = 0.4
_X0 = -51.2
_Y0 = -51.2
_ZC = 0.5 * (-5.0 + 3.0)
_BEV_W = 256
_BEV_H = 256
_B = 4
_M = _B * _BEV_H * _BEV_W
_CIN = 16
_COUT = 32

_BLK = 8000


def _mlp_body(xyz_ref, ptf_ref, b0_ref, w_ref, b_ref, h_ref, cell_ref):
    x = xyz_ref[:, 0]
    y = xyz_ref[:, 1]
    z = xyz_ref[:, 2]
    xi = jnp.clip(jnp.floor((x - _X0) / _PILLAR).astype(jnp.int32), 0, _BEV_W - 1)
    yi = jnp.clip(jnp.floor((y - _Y0) / _PILLAR).astype(jnp.int32), 0, _BEV_H - 1)
    bi = b0_ref[:, 0].astype(jnp.int32)
    cell_ref[0, 0, :] = bi * (_BEV_H * _BEV_W) + yi * _BEV_W + xi
    cx = (xi.astype(jnp.float32) + 0.5) * _PILLAR + _X0
    cy = (yi.astype(jnp.float32) + 0.5) * _PILLAR + _Y0
    fc = jnp.stack([x - cx, y - cy, z - _ZC], axis=1)
    gf = jnp.concatenate([fc, ptf_ref[...]], axis=1)
    h = jnp.dot(gf, w_ref[...], preferred_element_type=jnp.float32) + b_ref[...]
    h_ref[...] = jnp.maximum(h, 0.0)


def _mlp_features(xyz, bxyz0, pt_feature, W1, b1):
    n = xyz.shape[0]
    grid = n // _BLK
    return pl.pallas_call(
        _mlp_body,
        grid=(grid,),
        in_specs=[
            pl.BlockSpec((_BLK, 3), lambda i: (i, 0)),
            pl.BlockSpec((_BLK, _CIN), lambda i: (i, 0)),
            pl.BlockSpec((_BLK, 1), lambda i: (i, 0)),
            pl.BlockSpec((3 + _CIN, _COUT), lambda i: (0, 0)),
            pl.BlockSpec((_COUT,), lambda i: (0,)),
        ],
        out_specs=[
            pl.BlockSpec((_BLK, _COUT), lambda i: (i, 0)),
            pl.BlockSpec((1, 1, _BLK), lambda i: (i, 0, 0)),
        ],
        out_shape=[
            jax.ShapeDtypeStruct((n, _COUT), jnp.float32),
            jax.ShapeDtypeStruct((grid, 1, _BLK), jnp.int32),
        ],
    )(xyz, pt_feature, bxyz0, W1, b1)


def kernel(xyz, xyz_batch_cnt, pt_feature, bxyz, W1, b1):
    h, cells = _mlp_features(xyz, bxyz[:, :1], pt_feature, W1, b1)
    cells = cells.reshape(-1)
    dense = jnp.full((_M, _COUT), -jnp.inf, jnp.float32).at[cells].max(h)
    occ = dense[:, 0] != -jnp.inf
    rank = jnp.cumsum(occ.astype(jnp.int32)) - occ.astype(jnp.int32)
    pos = jnp.where(occ, rank, _M)
    cell_ids = jnp.arange(_M, dtype=jnp.int32)
    pf = jnp.zeros((_M, _COUT), jnp.float32).at[pos].set(
        jnp.where(occ[:, None], dense, 0.0), mode="drop")
    unq = jnp.full((_M,), -1, jnp.int32).at[pos].set(cell_ids, mode="drop")
    bb = unq // (_BEV_H * _BEV_W)
    rem = jnp.where(unq >= 0, unq % (_BEV_H * _BEV_W), 0)
    pillar_indices = jnp.stack([bb, rem // _BEV_W, rem % _BEV_W], axis=1)
    return pf, pillar_indices
